# baseline (speedup 1.0000x reference)
import functools

import jax
import jax.numpy as jnp
from jax import lax
from jax.experimental import pallas as pl
from jax.experimental.pallas import tpu as pltpu
from jax.experimental.pallas import tpu_sc as plsc

N = 10000
E = 160000
C = 256
CH = C // 2
K = 128
NCHUNK = 1344
EPAD = NCHUNK * K - E - N
NSUB = 16
NCORE = 2
CPT = NCHUNK // NSUB
CPW = NCHUNK // (NSUB * NCORE)
NACC = N + 8
NBUF = 2
ROWS_BIG, ROWS_SMALL = 632, 624

_mesh = lambda: plsc.VectorSubcoreMesh(core_axis_name="c", subcore_axis_name="s")


def _row_base(s):
    return ROWS_SMALL * s + 8 * jnp.minimum(s, 2)


@functools.partial(
    pl.kernel,
    out_type=jax.ShapeDtypeStruct((NCORE, 1, NACC), jnp.float32),
    mesh=_mesh(),
    scratch_types=[
        pltpu.VMEM((CPW, 1, K), jnp.int32),
        pltpu.VMEM((CPW, 1, K), jnp.int32),
        pltpu.VMEM((K,), jnp.float32),
        pltpu.VMEM((2000,), jnp.float32),
        pltpu.VMEM_SHARED((NACC,), jnp.float32),
        pltpu.SemaphoreType.DMA,
        pltpu.SemaphoreType.DMA,
    ],
)
def _deg_kernel(ed_hbm, out_hbm, ed_slab, dst_slab, ones_v, zbuf, acc,
                semd, semz):
    c = lax.axis_index("c")
    s = lax.axis_index("s")
    w = s * NCORE + c

    base = w * CPW
    pltpu.sync_copy(ed_hbm.at[pl.ds(base, CPW)], ed_slab)
    def unpack(i, _):
        sl = pl.ds((i % 8) * 16, 16)
        dst_slab[i // 8, 0, sl] = lax.shift_right_logical(ed_slab[i // 8, 0, sl], 14)
        return 0
    lax.fori_loop(0, CPW * (K // 16), unpack, 0)

    def fill_ones(i, _):
        ones_v[pl.ds(i * 16, 16)] = jnp.ones((16,), jnp.float32)
        return 0
    lax.fori_loop(0, K // 16, fill_ones, 0)
    def fill_z(i, _):
        zbuf[pl.ds(i * 16, 16)] = jnp.zeros((16,), jnp.float32)
        return 0
    lax.fori_loop(0, 125, fill_z, 0)
    @pl.when(s == 0)
    def _():
        def zfire(j, _):
            pltpu.async_copy(zbuf, acc.at[pl.ds(j * 2000, 2000)], semz)
            return 0
        lax.fori_loop(0, N // 2000, zfire, 0)
        def zdrain(j, _):
            pltpu.make_async_copy(zbuf, acc.at[pl.ds(0, 2000)], semz).wait()
            return 0
        lax.fori_loop(0, N // 2000, zdrain, 0)
    plsc.subcore_barrier()

    nch = CPW
    def fire(j, _):
        pltpu.async_copy(ones_v, acc.at[dst_slab.at[j, 0]], semd, add=True)
        return 0
    lax.fori_loop(0, nch, fire, 0)
    def drain(j, _):
        pltpu.make_async_copy(ones_v, acc.at[dst_slab.at[0, 0]], semd).wait()
        return 0
    lax.fori_loop(0, nch, drain, 0)
    plsc.subcore_barrier()

    @pl.when(s == 0)
    def _():
        pltpu.sync_copy(acc, out_hbm.at[c, 0])


@functools.partial(
    pl.kernel,
    out_type=jax.ShapeDtypeStruct((NCORE, N, CH), jnp.float32),
    mesh=_mesh(),
    scratch_types=[
        pltpu.VMEM((CPT, 1, K), jnp.int32),
        pltpu.VMEM((NBUF, 1, K), jnp.int32),
        pltpu.VMEM((NBUF, 1, K), jnp.int32),
        pltpu.VMEM((K, CH), jnp.float32),
        pltpu.VMEM((K, CH), jnp.float32),
        pltpu.VMEM((8, CH), jnp.float32),
        pltpu.VMEM_SHARED((NACC, CH), jnp.float32),
        pltpu.SemaphoreType.DMA,
        pltpu.SemaphoreType.DMA,
        pltpu.SemaphoreType.DMA,
        pltpu.SemaphoreType.DMA,
        pltpu.SemaphoreType.DMA,
    ],
)
def _agg_kernel(xs_hbm, ed_hbm, out_hbm,
                ed_slab, src_v, dst_v, r0, r1, zbuf, acc,
                g0, g1, s0, s1, semz):
    rows = [r0, r1]
    semg = [g0, g1]
    sems = [s0, s1]
    c = lax.axis_index("c")
    s = lax.axis_index("s")

    base = s * CPT
    pltpu.sync_copy(ed_hbm.at[pl.ds(base, CPT)], ed_slab)

    off = c * N
    def unpack(j, b):
        def go(i, _):
            sl = pl.ds(i * 16, 16)
            ed = ed_slab[j, 0, sl]
            src_v[b, 0, sl] = (ed & 0x3FFF) + off
            dst_v[b, 0, sl] = lax.shift_right_logical(ed, 14)
            return 0
        lax.fori_loop(0, K // 16, go, 0)

    def fill_z(i, _):
        zbuf[i // 8, pl.ds((i % 8) * 16, 16)] = jnp.zeros((16,), jnp.float32)
        return 0
    lax.fori_loop(0, 8 * (CH // 16), fill_z, 0)
    rbase = _row_base(s)
    nz = jnp.where(s < 2, ROWS_BIG // 8, ROWS_SMALL // 8)
    def zfire(j, _):
        pltpu.async_copy(zbuf, acc.at[pl.ds(rbase + j * 8, 8)], semz)
        return 0
    lax.fori_loop(0, nz, zfire, 0)
    def zdrain(j, _):
        pltpu.make_async_copy(zbuf, acc.at[pl.ds(rbase, 8)], semz).wait()
        return 0
    lax.fori_loop(0, nz, zdrain, 0)
    plsc.subcore_barrier()

    def gfire(j, b):
        pltpu.async_copy(xs_hbm.at[src_v.at[b, 0]], rows[b], semg[b])
    def gwait(b):
        pltpu.make_async_copy(xs_hbm.at[src_v.at[b, 0]], rows[b], semg[b]).wait()
    def sfire(b):
        pltpu.async_copy(rows[b], acc.at[dst_v.at[b, 0]], sems[b], add=True)
    def swait(b):
        pltpu.make_async_copy(rows[b], acc.at[dst_v.at[b, 0]], sems[b]).wait()

    unpack(0, 0)
    gfire(0, 0)
    unpack(1, 1)
    gfire(1, 1)
    gwait(0)
    sfire(0)
    def steady(g, _):
        for b in range(NBUF):
            j = g * NBUF + b
            swait(b)
            unpack(j, b)
            gfire(j, b)
            gwait(1 - b)
            sfire(1 - b)
        return 0
    lax.fori_loop(1, CPT // NBUF, steady, 0)
    gwait(1)
    sfire(1)
    swait(0)
    swait(1)
    plsc.subcore_barrier()

    rbig = pl.ds(rbase, ROWS_BIG)
    rsml = pl.ds(rbase, ROWS_SMALL)
    @pl.when(s < 2)
    def _():
        pltpu.sync_copy(acc.at[rbig], out_hbm.at[c, rbig])
    @pl.when(s >= 2)
    def _():
        pltpu.sync_copy(acc.at[rsml], out_hbm.at[c, rsml])


_TC_ROWS = 1000


def _dis_block(degp_ref):
    return lax.rsqrt(degp_ref[0] + degp_ref[1])


def _scale_body(degp_ref, x_ref, xs_ref):
    dis = _dis_block(degp_ref)
    xs = x_ref[...] * dis
    xs_ref[0] = xs[:, :CH]
    xs_ref[1] = xs[:, CH:]


def _scale(degp, x):
    grid = N // _TC_ROWS
    return pl.pallas_call(
        _scale_body,
        grid=(grid,),
        in_specs=[
            pl.BlockSpec((NCORE, _TC_ROWS, 1), lambda i: (0, i, 0)),
            pl.BlockSpec((_TC_ROWS, C), lambda i: (i, 0)),
        ],
        out_specs=pl.BlockSpec((NCORE, _TC_ROWS, CH), lambda i: (0, i, 0)),
        out_shape=jax.ShapeDtypeStruct((NCORE, N, CH), jnp.float32),
    )(degp, x)


def _final_body(degp_ref, zp_ref, wmu_ref, wls_ref, bmu_ref, bls_ref,
                mu_ref, ls_ref):
    dis = _dis_block(degp_ref)
    zlo = dis * zp_ref[0]
    zhi = dis * zp_ref[1]
    mu_ref[...] = (
        jnp.dot(zlo, wmu_ref[:CH, :], preferred_element_type=jnp.float32)
        + jnp.dot(zhi, wmu_ref[CH:, :], preferred_element_type=jnp.float32)
        + bmu_ref[...]
    )
    ls_ref[...] = (
        jnp.dot(zlo, wls_ref[:CH, :], preferred_element_type=jnp.float32)
        + jnp.dot(zhi, wls_ref[CH:, :], preferred_element_type=jnp.float32)
        + bls_ref[...]
    )


def _final(degp, zp, wmu, wls, bmu, bls):
    grid = N // _TC_ROWS
    return pl.pallas_call(
        _final_body,
        grid=(grid,),
        in_specs=[
            pl.BlockSpec((NCORE, _TC_ROWS, 1), lambda i: (0, i, 0)),
            pl.BlockSpec((NCORE, _TC_ROWS, CH), lambda i: (0, i, 0)),
            pl.BlockSpec((C, C), lambda i: (0, 0)),
            pl.BlockSpec((C, C), lambda i: (0, 0)),
            pl.BlockSpec((1, C), lambda i: (0, 0)),
            pl.BlockSpec((1, C), lambda i: (0, 0)),
        ],
        out_specs=[
            pl.BlockSpec((_TC_ROWS, C), lambda i: (i, 0)),
            pl.BlockSpec((_TC_ROWS, C), lambda i: (i, 0)),
        ],
        out_shape=[
            jax.ShapeDtypeStruct((N, C), jnp.float32),
            jax.ShapeDtypeStruct((N, C), jnp.float32),
        ],
    )(degp, zp, wmu, wls, bmu, bls)


def kernel(x, edge_index, W_mu, b_mu, W_logstd, b_logstd):
    src = edge_index[0].astype(jnp.int32)
    dst = edge_index[1].astype(jnp.int32)
    iota = jnp.arange(N, dtype=jnp.int32)
    ed = jnp.concatenate([
        src | (dst << 14),
        iota | (iota << 14),
        jnp.full((EPAD,), N << 14, jnp.int32),
    ]).reshape(NCHUNK, 1, K)

    degp = _deg_kernel(ed)[:, :, :N].reshape(NCORE, N, 1)
    xs2 = _scale(degp, x)
    zp = _agg_kernel(xs2.reshape(2 * N, CH), ed)
    mu, ls = _final(degp, zp, W_mu, W_logstd,
                    b_mu.reshape(1, C), b_logstd.reshape(1, C))
    return mu, ls


# speedup vs baseline: 1.5823x; 1.5823x over previous
import functools

import jax
import jax.numpy as jnp
from jax import lax
from jax.experimental import pallas as pl
from jax.experimental.pallas import tpu as pltpu
from jax.experimental.pallas import tpu_sc as plsc

N = 10000
E = 160000
C = 256
CH = C // 2
K = 128
NCHUNK = 1344
EPAD = NCHUNK * K - E - N
NSUB = 16
NCORE = 2
CPT = NCHUNK // NSUB
CPW = NCHUNK // (NSUB * NCORE)
NACC = N + 512
NBUF = 2
ROWS_BIG, ROWS_SMALL = 632, 624

_mesh = lambda: plsc.VectorSubcoreMesh(core_axis_name="c", subcore_axis_name="s")


def _row_base(s):
    return ROWS_SMALL * s + 8 * jnp.minimum(s, 2)


@functools.partial(
    pl.kernel,
    out_type=jax.ShapeDtypeStruct((NCORE, 1, NACC), jnp.float32),
    mesh=_mesh(),
    scratch_types=[
        pltpu.VMEM((CPW, 1, K), jnp.int32),
        pltpu.VMEM((CPW, 1, K), jnp.int32),
        pltpu.VMEM((K,), jnp.float32),
        pltpu.VMEM((2000,), jnp.float32),
        pltpu.VMEM_SHARED((NACC,), jnp.float32),
        pltpu.SemaphoreType.DMA,
        pltpu.SemaphoreType.DMA,
    ],
)
def _deg_kernel(ed_hbm, out_hbm, ed_slab, dst_slab, ones_v, zbuf, acc,
                semd, semz):
    c = lax.axis_index("c")
    s = lax.axis_index("s")
    w = s * NCORE + c

    base = w * CPW
    pltpu.sync_copy(ed_hbm.at[pl.ds(base, CPW)], ed_slab)
    def unpack(i, _):
        sl = pl.ds((i % 8) * 16, 16)
        dst_slab[i // 8, 0, sl] = lax.shift_right_logical(ed_slab[i // 8, 0, sl], 14)
        return 0
    lax.fori_loop(0, CPW * (K // 16), unpack, 0)

    def fill_ones(i, _):
        ones_v[pl.ds(i * 16, 16)] = jnp.ones((16,), jnp.float32)
        return 0
    lax.fori_loop(0, K // 16, fill_ones, 0)
    def fill_z(i, _):
        zbuf[pl.ds(i * 16, 16)] = jnp.zeros((16,), jnp.float32)
        return 0
    lax.fori_loop(0, 125, fill_z, 0)
    @pl.when(s == 0)
    def _():
        def zfire(j, _):
            pltpu.async_copy(zbuf, acc.at[pl.ds(j * 2000, 2000)], semz)
            return 0
        lax.fori_loop(0, N // 2000, zfire, 0)
        def zdrain(j, _):
            pltpu.make_async_copy(zbuf, acc.at[pl.ds(0, 2000)], semz).wait()
            return 0
        lax.fori_loop(0, N // 2000, zdrain, 0)
    plsc.subcore_barrier()

    nch = CPW
    def fire(j, _):
        pltpu.async_copy(ones_v, acc.at[dst_slab.at[j, 0]], semd, add=True)
        return 0
    lax.fori_loop(0, nch, fire, 0)
    def drain(j, _):
        pltpu.make_async_copy(ones_v, acc.at[dst_slab.at[0, 0]], semd).wait()
        return 0
    lax.fori_loop(0, nch, drain, 0)
    plsc.subcore_barrier()

    @pl.when(s == 0)
    def _():
        pltpu.sync_copy(acc, out_hbm.at[c, 0])


@functools.partial(
    pl.kernel,
    out_type=jax.ShapeDtypeStruct((NCORE, N, CH), jnp.float32),
    mesh=_mesh(),
    scratch_types=[
        pltpu.VMEM((CPT, 1, K), jnp.int32),
        pltpu.VMEM((NBUF, 1, K), jnp.int32),
        pltpu.VMEM((NBUF, 1, K), jnp.int32),
        pltpu.VMEM((K, CH), jnp.float32),
        pltpu.VMEM((K, CH), jnp.float32),
        pltpu.VMEM((8, CH), jnp.float32),
        pltpu.VMEM_SHARED((NACC, CH), jnp.float32),
        pltpu.SemaphoreType.DMA,
        pltpu.SemaphoreType.DMA,
        pltpu.SemaphoreType.DMA,
        pltpu.SemaphoreType.DMA,
        pltpu.SemaphoreType.DMA,
    ],
)
def _agg_kernel(xs_hbm, ed_hbm, out_hbm,
                ed_slab, src_v, dst_v, r0, r1, zbuf, acc,
                g0, g1, s0, s1, semz):
    rows = [r0, r1]
    semg = [g0, g1]
    sems = [s0, s1]
    c = lax.axis_index("c")
    s = lax.axis_index("s")

    base = s * CPT
    pltpu.sync_copy(ed_hbm.at[pl.ds(base, CPT)], ed_slab)

    off = c * N
    def unpack(j, b):
        def go(i, _):
            sl = pl.ds(i * 16, 16)
            ed = ed_slab[j, 0, sl]
            src_v[b, 0, sl] = (ed & 0x3FFF) + off
            dst_v[b, 0, sl] = lax.shift_right_logical(ed, 14)
            return 0
        lax.fori_loop(0, K // 16, go, 0)

    def fill_z(i, _):
        zbuf[i // 8, pl.ds((i % 8) * 16, 16)] = jnp.zeros((16,), jnp.float32)
        return 0
    lax.fori_loop(0, 8 * (CH // 16), fill_z, 0)
    rbase = _row_base(s)
    nz = jnp.where(s < 2, ROWS_BIG // 8, ROWS_SMALL // 8)
    def zfire(j, _):
        pltpu.async_copy(zbuf, acc.at[pl.ds(rbase + j * 8, 8)], semz)
        return 0
    lax.fori_loop(0, nz, zfire, 0)
    def zdrain(j, _):
        pltpu.make_async_copy(zbuf, acc.at[pl.ds(rbase, 8)], semz).wait()
        return 0
    lax.fori_loop(0, nz, zdrain, 0)
    plsc.subcore_barrier()

    def gfire(j, b):
        pltpu.async_copy(xs_hbm.at[src_v.at[b, 0]], rows[b], semg[b])
    def gwait(b):
        pltpu.make_async_copy(xs_hbm.at[src_v.at[b, 0]], rows[b], semg[b]).wait()
    def sfire(b):
        pltpu.async_copy(rows[b], acc.at[dst_v.at[b, 0]], sems[b], add=True)
    def swait(b):
        pltpu.make_async_copy(rows[b], acc.at[dst_v.at[b, 0]], sems[b]).wait()

    unpack(0, 0)
    gfire(0, 0)
    unpack(1, 1)
    gfire(1, 1)
    gwait(0)
    sfire(0)
    def steady(g, _):
        for b in range(NBUF):
            j = g * NBUF + b
            swait(b)
            unpack(j, b)
            gfire(j, b)
            gwait(1 - b)
            sfire(1 - b)
        return 0
    lax.fori_loop(1, CPT // NBUF, steady, 0)
    gwait(1)
    sfire(1)
    swait(0)
    swait(1)
    plsc.subcore_barrier()

    rbig = pl.ds(rbase, ROWS_BIG)
    rsml = pl.ds(rbase, ROWS_SMALL)
    @pl.when(s < 2)
    def _():
        pltpu.sync_copy(acc.at[rbig], out_hbm.at[c, rbig])
    @pl.when(s >= 2)
    def _():
        pltpu.sync_copy(acc.at[rsml], out_hbm.at[c, rsml])


_TC_ROWS = 1000


def _dis_block(degp_ref):
    return lax.rsqrt(degp_ref[0] + degp_ref[1])


def _scale_body(degp_ref, x_ref, xs_ref):
    dis = _dis_block(degp_ref)
    xs = x_ref[...] * dis
    xs_ref[0] = xs[:, :CH]
    xs_ref[1] = xs[:, CH:]


def _scale(degp, x):
    grid = N // _TC_ROWS
    return pl.pallas_call(
        _scale_body,
        grid=(grid,),
        in_specs=[
            pl.BlockSpec((NCORE, _TC_ROWS, 1), lambda i: (0, i, 0)),
            pl.BlockSpec((_TC_ROWS, C), lambda i: (i, 0)),
        ],
        out_specs=pl.BlockSpec((NCORE, _TC_ROWS, CH), lambda i: (0, i, 0)),
        out_shape=jax.ShapeDtypeStruct((NCORE, N, CH), jnp.float32),
    )(degp, x)


def _final_body(degp_ref, zp_ref, wmu_ref, wls_ref, bmu_ref, bls_ref,
                mu_ref, ls_ref):
    dis = _dis_block(degp_ref)
    zlo = dis * zp_ref[0]
    zhi = dis * zp_ref[1]
    mu_ref[...] = (
        jnp.dot(zlo, wmu_ref[:CH, :], preferred_element_type=jnp.float32)
        + jnp.dot(zhi, wmu_ref[CH:, :], preferred_element_type=jnp.float32)
        + bmu_ref[...]
    )
    ls_ref[...] = (
        jnp.dot(zlo, wls_ref[:CH, :], preferred_element_type=jnp.float32)
        + jnp.dot(zhi, wls_ref[CH:, :], preferred_element_type=jnp.float32)
        + bls_ref[...]
    )


def _final(degp, zp, wmu, wls, bmu, bls):
    grid = N // _TC_ROWS
    return pl.pallas_call(
        _final_body,
        grid=(grid,),
        in_specs=[
            pl.BlockSpec((NCORE, _TC_ROWS, 1), lambda i: (0, i, 0)),
            pl.BlockSpec((NCORE, _TC_ROWS, CH), lambda i: (0, i, 0)),
            pl.BlockSpec((C, C), lambda i: (0, 0)),
            pl.BlockSpec((C, C), lambda i: (0, 0)),
            pl.BlockSpec((1, C), lambda i: (0, 0)),
            pl.BlockSpec((1, C), lambda i: (0, 0)),
        ],
        out_specs=[
            pl.BlockSpec((_TC_ROWS, C), lambda i: (i, 0)),
            pl.BlockSpec((_TC_ROWS, C), lambda i: (i, 0)),
        ],
        out_shape=[
            jax.ShapeDtypeStruct((N, C), jnp.float32),
            jax.ShapeDtypeStruct((N, C), jnp.float32),
        ],
    )(degp, zp, wmu, wls, bmu, bls)


def kernel(x, edge_index, W_mu, b_mu, W_logstd, b_logstd):
    src = edge_index[0].astype(jnp.int32)
    dst = edge_index[1].astype(jnp.int32)
    iota = jnp.arange(N, dtype=jnp.int32)
    ipad = jnp.arange(EPAD, dtype=jnp.int32)
    ed = jnp.concatenate([
        src | (dst << 14),
        iota | (iota << 14),
        (ipad & 0x1FFF) | ((N + (ipad & 511)) << 14),
    ]).reshape(NCHUNK, 1, K)

    degp = _deg_kernel(ed)[:, :, :N].reshape(NCORE, N, 1)
    xs2 = _scale(degp, x)
    zp = _agg_kernel(xs2.reshape(2 * N, CH), ed)
    mu, ls = _final(degp, zp, W_mu, W_logstd,
                    b_mu.reshape(1, C), b_logstd.reshape(1, C))
    return mu, ls


# speedup vs baseline: 1.7208x; 1.0875x over previous
import functools

import jax
import jax.numpy as jnp
from jax import lax
from jax.experimental import pallas as pl
from jax.experimental.pallas import tpu as pltpu
from jax.experimental.pallas import tpu_sc as plsc

N = 10000
E = 160000
C = 256
CH = C // 2
K = 128
NCHUNK = 1344
NREAL = E // K
NEXTRA = NCHUNK - NREAL
EPAD = NCHUNK * K - E - N
NSUB = 16
NCORE = 2
CPT = NCHUNK // NSUB
CPW = NCHUNK // (NSUB * NCORE)
NACC = N + 512
NBUF = 2
ROWS_BIG, ROWS_SMALL = 632, 624

_mesh = lambda: plsc.VectorSubcoreMesh(core_axis_name="c", subcore_axis_name="s")


def _row_base(s):
    return ROWS_SMALL * s + 8 * jnp.minimum(s, 2)


@functools.partial(
    pl.kernel,
    out_type=jax.ShapeDtypeStruct((NCORE, 1, NACC), jnp.float32),
    mesh=_mesh(),
    scratch_types=[
        pltpu.VMEM((CPW, 1, K), jnp.int32),
        pltpu.VMEM((CPW, 1, K), jnp.int32),
        pltpu.VMEM((K,), jnp.float32),
        pltpu.VMEM((2000,), jnp.float32),
        pltpu.VMEM_SHARED((NACC,), jnp.float32),
        pltpu.SemaphoreType.DMA,
        pltpu.SemaphoreType.DMA,
    ],
)
def _deg_kernel(ed_hbm, ex_hbm, out_hbm, ed_slab, dst_slab, ones_v, zbuf, acc,
                semd, semz):
    c = lax.axis_index("c")
    s = lax.axis_index("s")
    w = s * NCORE + c

    base = w * CPW
    BW = NREAL // CPW
    RREM = NREAL - BW * CPW
    @pl.when(w < BW)
    def _():
        pltpu.sync_copy(ed_hbm.at[pl.ds(base, CPW)], ed_slab)
    @pl.when(w == BW)
    def _():
        pltpu.sync_copy(ed_hbm.at[pl.ds(BW * CPW, RREM)],
                        ed_slab.at[pl.ds(0, RREM)])
        pltpu.sync_copy(ex_hbm.at[pl.ds(0, CPW - RREM)],
                        ed_slab.at[pl.ds(RREM, CPW - RREM)])
    @pl.when(w > BW)
    def _():
        pltpu.sync_copy(ex_hbm.at[pl.ds(base - NREAL, CPW)], ed_slab)
    def unpack(i, _):
        sl = pl.ds((i % 8) * 16, 16)
        dst_slab[i // 8, 0, sl] = lax.shift_right_logical(ed_slab[i // 8, 0, sl], 14)
        return 0
    lax.fori_loop(0, CPW * (K // 16), unpack, 0)

    def fill_ones(i, _):
        ones_v[pl.ds(i * 16, 16)] = jnp.ones((16,), jnp.float32)
        return 0
    lax.fori_loop(0, K // 16, fill_ones, 0)
    def fill_z(i, _):
        zbuf[pl.ds(i * 16, 16)] = jnp.zeros((16,), jnp.float32)
        return 0
    lax.fori_loop(0, 125, fill_z, 0)
    @pl.when(s == 0)
    def _():
        def zfire(j, _):
            pltpu.async_copy(zbuf, acc.at[pl.ds(j * 2000, 2000)], semz)
            return 0
        lax.fori_loop(0, N // 2000, zfire, 0)
        def zdrain(j, _):
            pltpu.make_async_copy(zbuf, acc.at[pl.ds(0, 2000)], semz).wait()
            return 0
        lax.fori_loop(0, N // 2000, zdrain, 0)
    plsc.subcore_barrier()

    nch = CPW
    def fire(j, _):
        pltpu.async_copy(ones_v, acc.at[dst_slab.at[j, 0]], semd, add=True)
        return 0
    lax.fori_loop(0, nch, fire, 0)
    def drain(j, _):
        pltpu.make_async_copy(ones_v, acc.at[dst_slab.at[0, 0]], semd).wait()
        return 0
    lax.fori_loop(0, nch, drain, 0)
    plsc.subcore_barrier()

    @pl.when(s == 0)
    def _():
        pltpu.sync_copy(acc, out_hbm.at[c, 0])


@functools.partial(
    pl.kernel,
    out_type=jax.ShapeDtypeStruct((NCORE, N, CH), jnp.float32),
    mesh=_mesh(),
    scratch_types=[
        pltpu.VMEM((CPT, 1, K), jnp.int32),
        pltpu.VMEM((NBUF, 1, K), jnp.int32),
        pltpu.VMEM((NBUF, 1, K), jnp.int32),
        pltpu.VMEM((K, CH), jnp.float32),
        pltpu.VMEM((K, CH), jnp.float32),
        pltpu.VMEM((8, CH), jnp.float32),
        pltpu.VMEM_SHARED((NACC, CH), jnp.float32),
        pltpu.SemaphoreType.DMA,
        pltpu.SemaphoreType.DMA,
        pltpu.SemaphoreType.DMA,
        pltpu.SemaphoreType.DMA,
        pltpu.SemaphoreType.DMA,
    ],
)
def _agg_kernel(xs_hbm, ed_hbm, ex_hbm, out_hbm,
                ed_slab, src_v, dst_v, r0, r1, zbuf, acc,
                g0, g1, s0, s1, semz):
    rows = [r0, r1]
    semg = [g0, g1]
    sems = [s0, s1]
    c = lax.axis_index("c")
    s = lax.axis_index("s")

    base = s * CPT
    BT = NREAL // CPT
    TREM = NREAL - BT * CPT
    @pl.when(s < BT)
    def _():
        pltpu.sync_copy(ed_hbm.at[pl.ds(base, CPT)], ed_slab)
    @pl.when(s == BT)
    def _():
        pltpu.sync_copy(ed_hbm.at[pl.ds(BT * CPT, TREM)],
                        ed_slab.at[pl.ds(0, TREM)])
        pltpu.sync_copy(ex_hbm.at[pl.ds(0, CPT - TREM)],
                        ed_slab.at[pl.ds(TREM, CPT - TREM)])
    @pl.when(s > BT)
    def _():
        pltpu.sync_copy(ex_hbm.at[pl.ds(base - NREAL, CPT)], ed_slab)

    off = c * N
    def unpack(j, b):
        def go(i, _):
            sl = pl.ds(i * 16, 16)
            ed = ed_slab[j, 0, sl]
            src_v[b, 0, sl] = (ed & 0x3FFF) + off
            dst_v[b, 0, sl] = lax.shift_right_logical(ed, 14)
            return 0
        lax.fori_loop(0, K // 16, go, 0)

    def fill_z(i, _):
        zbuf[i // 8, pl.ds((i % 8) * 16, 16)] = jnp.zeros((16,), jnp.float32)
        return 0
    lax.fori_loop(0, 8 * (CH // 16), fill_z, 0)
    rbase = _row_base(s)
    nz = jnp.where(s < 2, ROWS_BIG // 8, ROWS_SMALL // 8)
    def zfire(j, _):
        pltpu.async_copy(zbuf, acc.at[pl.ds(rbase + j * 8, 8)], semz)
        return 0
    lax.fori_loop(0, nz, zfire, 0)
    def zdrain(j, _):
        pltpu.make_async_copy(zbuf, acc.at[pl.ds(rbase, 8)], semz).wait()
        return 0
    lax.fori_loop(0, nz, zdrain, 0)
    plsc.subcore_barrier()

    def gfire(j, b):
        pltpu.async_copy(xs_hbm.at[src_v.at[b, 0]], rows[b], semg[b])
    def gwait(b):
        pltpu.make_async_copy(xs_hbm.at[src_v.at[b, 0]], rows[b], semg[b]).wait()
    def sfire(b):
        pltpu.async_copy(rows[b], acc.at[dst_v.at[b, 0]], sems[b], add=True)
    def swait(b):
        pltpu.make_async_copy(rows[b], acc.at[dst_v.at[b, 0]], sems[b]).wait()

    unpack(0, 0)
    gfire(0, 0)
    unpack(1, 1)
    gfire(1, 1)
    gwait(0)
    sfire(0)
    def steady(g, _):
        for b in range(NBUF):
            j = g * NBUF + b
            swait(b)
            unpack(j, b)
            gfire(j, b)
            gwait(1 - b)
            sfire(1 - b)
        return 0
    lax.fori_loop(1, CPT // NBUF, steady, 0)
    gwait(1)
    sfire(1)
    swait(0)
    swait(1)
    plsc.subcore_barrier()

    rbig = pl.ds(rbase, ROWS_BIG)
    rsml = pl.ds(rbase, ROWS_SMALL)
    @pl.when(s < 2)
    def _():
        pltpu.sync_copy(acc.at[rbig], out_hbm.at[c, rbig])
    @pl.when(s >= 2)
    def _():
        pltpu.sync_copy(acc.at[rsml], out_hbm.at[c, rsml])


_TC_ROWS = 1000


def _trans_body(degp_ref, dis_ref):
    ident = jnp.eye(_TC_ROWS, dtype=jnp.float32)
    for chunk in range(N // _TC_ROWS):
        seg = (degp_ref[0, :, pl.ds(chunk * _TC_ROWS, _TC_ROWS)]
               + degp_ref[1, :, pl.ds(chunk * _TC_ROWS, _TC_ROWS)])
        col = lax.dot_general(ident, lax.rsqrt(seg),
                              (((1,), (1,)), ((), ())),
                              preferred_element_type=jnp.float32)
        dis_ref[pl.ds(chunk * _TC_ROWS, _TC_ROWS), :] = col


def _trans(degp):
    return pl.pallas_call(
        _trans_body,
        out_shape=jax.ShapeDtypeStruct((N, 1), jnp.float32),
    )(degp)


def _scale_body(dis_ref, x_ref, xs_ref):
    xs = x_ref[...] * dis_ref[...]
    xs_ref[0] = xs[:, :CH]
    xs_ref[1] = xs[:, CH:]


def _scale(dis, x):
    grid = N // _TC_ROWS
    return pl.pallas_call(
        _scale_body,
        grid=(grid,),
        in_specs=[
            pl.BlockSpec((_TC_ROWS, 1), lambda i: (i, 0)),
            pl.BlockSpec((_TC_ROWS, C), lambda i: (i, 0)),
        ],
        out_specs=pl.BlockSpec((NCORE, _TC_ROWS, CH), lambda i: (0, i, 0)),
        out_shape=jax.ShapeDtypeStruct((NCORE, N, CH), jnp.float32),
    )(dis, x)


def _final_body(dis_ref, zp_ref, wmu_ref, wls_ref, bmu_ref, bls_ref,
                mu_ref, ls_ref):
    dis = dis_ref[...]
    zlo = dis * zp_ref[0]
    zhi = dis * zp_ref[1]
    mu_ref[...] = (
        jnp.dot(zlo, wmu_ref[:CH, :], preferred_element_type=jnp.float32)
        + jnp.dot(zhi, wmu_ref[CH:, :], preferred_element_type=jnp.float32)
        + bmu_ref[...]
    )
    ls_ref[...] = (
        jnp.dot(zlo, wls_ref[:CH, :], preferred_element_type=jnp.float32)
        + jnp.dot(zhi, wls_ref[CH:, :], preferred_element_type=jnp.float32)
        + bls_ref[...]
    )


def _final(dis, zp, wmu, wls, bmu, bls):
    grid = N // _TC_ROWS
    return pl.pallas_call(
        _final_body,
        grid=(grid,),
        in_specs=[
            pl.BlockSpec((_TC_ROWS, 1), lambda i: (i, 0)),
            pl.BlockSpec((NCORE, _TC_ROWS, CH), lambda i: (0, i, 0)),
            pl.BlockSpec((C, C), lambda i: (0, 0)),
            pl.BlockSpec((C, C), lambda i: (0, 0)),
            pl.BlockSpec((1, C), lambda i: (0, 0)),
            pl.BlockSpec((1, C), lambda i: (0, 0)),
        ],
        out_specs=[
            pl.BlockSpec((_TC_ROWS, C), lambda i: (i, 0)),
            pl.BlockSpec((_TC_ROWS, C), lambda i: (i, 0)),
        ],
        out_shape=[
            jax.ShapeDtypeStruct((N, C), jnp.float32),
            jax.ShapeDtypeStruct((N, C), jnp.float32),
        ],
    )(dis, zp, wmu, wls, bmu, bls)


def kernel(x, edge_index, W_mu, b_mu, W_logstd, b_logstd):
    src = edge_index[0].astype(jnp.int32)
    dst = edge_index[1].astype(jnp.int32)
    ed = (src | (dst << 14)).reshape(NREAL, 1, K)
    iota = jnp.arange(N, dtype=jnp.int32)
    ipad = jnp.arange(EPAD, dtype=jnp.int32)
    ex = jnp.concatenate([
        iota | (iota << 14),
        (ipad & 0x1FFF) | ((N + (ipad & 511)) << 14),
    ]).reshape(NEXTRA, 1, K)

    degp = _deg_kernel(ed, ex)
    dis = _trans(degp)
    xs2 = _scale(dis, x)
    zp = _agg_kernel(xs2.reshape(2 * N, CH), ed, ex)
    mu, ls = _final(dis, zp, W_mu, W_logstd,
                    b_mu.reshape(1, C), b_logstd.reshape(1, C))
    return mu, ls


# speedup vs baseline: 1.7342x; 1.0078x over previous
import functools

import jax
import jax.numpy as jnp
from jax import lax
from jax.experimental import pallas as pl
from jax.experimental.pallas import tpu as pltpu
from jax.experimental.pallas import tpu_sc as plsc

N = 10000
E = 160000
C = 256
CH = C // 2
K = 128
NCHUNK = 1344
NREAL = E // K
NEXTRA = NCHUNK - NREAL
EPAD = NCHUNK * K - E - N
NSUB = 16
NCORE = 2
CPT = NCHUNK // NSUB
CPW = NCHUNK // (NSUB * NCORE)
NACC = N + 512
NBUF = 2
ROWS_BIG, ROWS_SMALL = 632, 624

_mesh = lambda: plsc.VectorSubcoreMesh(core_axis_name="c", subcore_axis_name="s")


def _row_base(s):
    return ROWS_SMALL * s + 8 * jnp.minimum(s, 2)


@functools.partial(
    pl.kernel,
    out_type=jax.ShapeDtypeStruct((NCORE, 1, NACC), jnp.float32),
    mesh=_mesh(),
    scratch_types=[
        pltpu.VMEM((CPW, 1, K), jnp.int32),
        pltpu.VMEM((CPW, 1, K), jnp.int32),
        pltpu.VMEM((K,), jnp.float32),
        pltpu.VMEM((2000,), jnp.float32),
        pltpu.VMEM_SHARED((NACC,), jnp.float32),
        pltpu.SemaphoreType.DMA,
        pltpu.SemaphoreType.DMA,
    ],
)
def _deg_kernel(ed_hbm, ex_hbm, out_hbm, ed_slab, dst_slab, ones_v, zbuf, acc,
                semd, semz):
    c = lax.axis_index("c")
    s = lax.axis_index("s")
    w = s * NCORE + c

    base = w * CPW
    BW = NREAL // CPW
    RREM = NREAL - BW * CPW
    @pl.when(w < BW)
    def _():
        pltpu.sync_copy(ed_hbm.at[pl.ds(base, CPW)], ed_slab)
    @pl.when(w == BW)
    def _():
        pltpu.sync_copy(ed_hbm.at[pl.ds(BW * CPW, RREM)],
                        ed_slab.at[pl.ds(0, RREM)])
        pltpu.sync_copy(ex_hbm.at[pl.ds(0, CPW - RREM)],
                        ed_slab.at[pl.ds(RREM, CPW - RREM)])
    @pl.when(w > BW)
    def _():
        pltpu.sync_copy(ex_hbm.at[pl.ds(base - NREAL, CPW)], ed_slab)
    def unpack(i, _):
        sl = pl.ds((i % 8) * 16, 16)
        dst_slab[i // 8, 0, sl] = lax.shift_right_logical(ed_slab[i // 8, 0, sl], 14)
        return 0
    lax.fori_loop(0, CPW * (K // 16), unpack, 0)

    def fill_ones(i, _):
        ones_v[pl.ds(i * 16, 16)] = jnp.ones((16,), jnp.float32)
        return 0
    lax.fori_loop(0, K // 16, fill_ones, 0)
    def fill_z(i, _):
        zbuf[pl.ds(i * 16, 16)] = jnp.zeros((16,), jnp.float32)
        return 0
    lax.fori_loop(0, 125, fill_z, 0)
    @pl.when(s == 0)
    def _():
        def zfire(j, _):
            pltpu.async_copy(zbuf, acc.at[pl.ds(j * 2000, 2000)], semz)
            return 0
        lax.fori_loop(0, N // 2000, zfire, 0)
        def zdrain(j, _):
            pltpu.make_async_copy(zbuf, acc.at[pl.ds(0, 2000)], semz).wait()
            return 0
        lax.fori_loop(0, N // 2000, zdrain, 0)
    plsc.subcore_barrier()

    nch = CPW
    def fire(j, _):
        pltpu.async_copy(ones_v, acc.at[dst_slab.at[j, 0]], semd, add=True)
        return 0
    lax.fori_loop(0, nch, fire, 0)
    def drain(j, _):
        pltpu.make_async_copy(ones_v, acc.at[dst_slab.at[0, 0]], semd).wait()
        return 0
    lax.fori_loop(0, nch, drain, 0)
    plsc.subcore_barrier()

    @pl.when(s == 0)
    def _():
        pltpu.sync_copy(acc, out_hbm.at[c, 0])


@functools.partial(
    pl.kernel,
    out_type=jax.ShapeDtypeStruct((NCORE, N, CH), jnp.float32),
    mesh=_mesh(),
    scratch_types=[
        pltpu.VMEM((CPT, 1, K), jnp.int32),
        pltpu.VMEM((NBUF, 1, K), jnp.int32),
        pltpu.VMEM((NBUF, 1, K), jnp.int32),
        pltpu.VMEM((K, CH), jnp.float32),
        pltpu.VMEM((K, CH), jnp.float32),
        pltpu.VMEM((8, CH), jnp.float32),
        pltpu.VMEM_SHARED((NACC, CH), jnp.float32),
        pltpu.SemaphoreType.DMA,
        pltpu.SemaphoreType.DMA,
        pltpu.SemaphoreType.DMA,
        pltpu.SemaphoreType.DMA,
        pltpu.SemaphoreType.DMA,
    ],
)
def _agg_kernel(xs_hbm, ed_hbm, ex_hbm, out_hbm,
                ed_slab, src_v, dst_v, r0, r1, zbuf, acc,
                g0, g1, s0, s1, semz):
    rows = [r0, r1]
    semg = [g0, g1]
    sems = [s0, s1]
    c = lax.axis_index("c")
    s = lax.axis_index("s")

    base = s * CPT
    BT = NREAL // CPT
    TREM = NREAL - BT * CPT
    @pl.when(s < BT)
    def _():
        pltpu.sync_copy(ed_hbm.at[pl.ds(base, CPT)], ed_slab)
    @pl.when(s == BT)
    def _():
        pltpu.sync_copy(ed_hbm.at[pl.ds(BT * CPT, TREM)],
                        ed_slab.at[pl.ds(0, TREM)])
        pltpu.sync_copy(ex_hbm.at[pl.ds(0, CPT - TREM)],
                        ed_slab.at[pl.ds(TREM, CPT - TREM)])
    @pl.when(s > BT)
    def _():
        pltpu.sync_copy(ex_hbm.at[pl.ds(base - NREAL, CPT)], ed_slab)

    off = c * N
    def unpack(j, b):
        def go(i, _):
            sl = pl.ds(i * 16, 16)
            ed = ed_slab[j, 0, sl]
            src_v[b, 0, sl] = (ed & 0x3FFF) + off
            dst_v[b, 0, sl] = lax.shift_right_logical(ed, 14)
            return 0
        lax.fori_loop(0, K // 16, go, 0)

    def fill_z(i, _):
        zbuf[i // 8, pl.ds((i % 8) * 16, 16)] = jnp.zeros((16,), jnp.float32)
        return 0
    lax.fori_loop(0, 8 * (CH // 16), fill_z, 0)
    rbase = _row_base(s)
    nz = jnp.where(s < 2, ROWS_BIG // 8, ROWS_SMALL // 8)
    def zfire(j, _):
        pltpu.async_copy(zbuf, acc.at[pl.ds(rbase + j * 8, 8)], semz)
        return 0
    lax.fori_loop(0, nz, zfire, 0)
    def zdrain(j, _):
        pltpu.make_async_copy(zbuf, acc.at[pl.ds(rbase, 8)], semz).wait()
        return 0
    lax.fori_loop(0, nz, zdrain, 0)
    plsc.subcore_barrier()

    def gfire(j, b):
        pltpu.async_copy(xs_hbm.at[src_v.at[b, 0]], rows[b], semg[b])
    def gwait(b):
        pltpu.make_async_copy(xs_hbm.at[src_v.at[b, 0]], rows[b], semg[b]).wait()
    def sfire(b):
        pltpu.async_copy(rows[b], acc.at[dst_v.at[b, 0]], sems[b], add=True)
    def swait(b):
        pltpu.make_async_copy(rows[b], acc.at[dst_v.at[b, 0]], sems[b]).wait()

    unpack(0, 0)
    gfire(0, 0)
    unpack(1, 1)
    gfire(1, 1)
    gwait(0)
    sfire(0)
    def steady(g, _):
        for b in range(NBUF):
            j = g * NBUF + b
            swait(b)
            unpack(j, b)
            gfire(j, b)
            gwait(1 - b)
            sfire(1 - b)
        return 0
    lax.fori_loop(1, CPT // NBUF, steady, 0)
    gwait(1)
    sfire(1)
    swait(0)
    swait(1)
    plsc.subcore_barrier()

    rbig = pl.ds(rbase, ROWS_BIG)
    rsml = pl.ds(rbase, ROWS_SMALL)
    @pl.when(s < 2)
    def _():
        pltpu.sync_copy(acc.at[rbig], out_hbm.at[c, rbig])
    @pl.when(s >= 2)
    def _():
        pltpu.sync_copy(acc.at[rsml], out_hbm.at[c, rsml])


_TC_ROWS = 1000


def _trans_body(degp_ref, dis_ref):
    ident = jnp.eye(_TC_ROWS, dtype=jnp.float32)
    for chunk in range(N // _TC_ROWS):
        seg = (degp_ref[0, :, pl.ds(chunk * _TC_ROWS, _TC_ROWS)]
               + degp_ref[1, :, pl.ds(chunk * _TC_ROWS, _TC_ROWS)])
        col = lax.dot_general(ident, lax.rsqrt(seg),
                              (((1,), (1,)), ((), ())),
                              preferred_element_type=jnp.float32)
        dis_ref[pl.ds(chunk * _TC_ROWS, _TC_ROWS), :] = col


def _trans(degp):
    return pl.pallas_call(
        _trans_body,
        out_shape=jax.ShapeDtypeStruct((N, 1), jnp.float32),
    )(degp)


def _scale_body(dis_ref, x_ref, xs_ref):
    xs = x_ref[...] * dis_ref[...]
    xs_ref[0] = xs[:, :CH]
    xs_ref[1] = xs[:, CH:]


def _scale(dis, x):
    grid = N // _TC_ROWS
    return pl.pallas_call(
        _scale_body,
        grid=(grid,),
        in_specs=[
            pl.BlockSpec((_TC_ROWS, 1), lambda i: (i, 0)),
            pl.BlockSpec((_TC_ROWS, C), lambda i: (i, 0)),
        ],
        out_specs=pl.BlockSpec((NCORE, _TC_ROWS, CH), lambda i: (0, i, 0)),
        out_shape=jax.ShapeDtypeStruct((NCORE, N, CH), jnp.float32),
    )(dis, x)


def _final_body(dis_ref, zp_ref, wmu_ref, wls_ref, bmu_ref, bls_ref,
                mu_ref, ls_ref):
    dis = dis_ref[...]
    z = jnp.concatenate([dis * zp_ref[0], dis * zp_ref[1]],
                        axis=1).astype(jnp.bfloat16)
    mu_ref[...] = jnp.dot(z, wmu_ref[...].astype(jnp.bfloat16),
                          preferred_element_type=jnp.float32) + bmu_ref[...]
    ls_ref[...] = jnp.dot(z, wls_ref[...].astype(jnp.bfloat16),
                          preferred_element_type=jnp.float32) + bls_ref[...]


def _final(dis, zp, wmu, wls, bmu, bls):
    grid = N // _TC_ROWS
    return pl.pallas_call(
        _final_body,
        grid=(grid,),
        in_specs=[
            pl.BlockSpec((_TC_ROWS, 1), lambda i: (i, 0)),
            pl.BlockSpec((NCORE, _TC_ROWS, CH), lambda i: (0, i, 0)),
            pl.BlockSpec((C, C), lambda i: (0, 0)),
            pl.BlockSpec((C, C), lambda i: (0, 0)),
            pl.BlockSpec((1, C), lambda i: (0, 0)),
            pl.BlockSpec((1, C), lambda i: (0, 0)),
        ],
        out_specs=[
            pl.BlockSpec((_TC_ROWS, C), lambda i: (i, 0)),
            pl.BlockSpec((_TC_ROWS, C), lambda i: (i, 0)),
        ],
        out_shape=[
            jax.ShapeDtypeStruct((N, C), jnp.float32),
            jax.ShapeDtypeStruct((N, C), jnp.float32),
        ],
    )(dis, zp, wmu, wls, bmu, bls)


def kernel(x, edge_index, W_mu, b_mu, W_logstd, b_logstd):
    src = edge_index[0].astype(jnp.int32)
    dst = edge_index[1].astype(jnp.int32)
    ed = (src | (dst << 14)).reshape(NREAL, 1, K)
    iota = jnp.arange(N, dtype=jnp.int32)
    ipad = jnp.arange(EPAD, dtype=jnp.int32)
    ex = jnp.concatenate([
        iota | (iota << 14),
        (ipad & 0x1FFF) | ((N + (ipad & 511)) << 14),
    ]).reshape(NEXTRA, 1, K)

    degp = _deg_kernel(ed, ex)
    dis = _trans(degp)
    xs2 = _scale(dis, x)
    zp = _agg_kernel(xs2.reshape(2 * N, CH), ed, ex)
    mu, ls = _final(dis, zp, W_mu, W_logstd,
                    b_mu.reshape(1, C), b_logstd.reshape(1, C))
    return mu, ls


# speedup vs baseline: 1.7474x; 1.0076x over previous
import functools

import jax
import jax.numpy as jnp
from jax import lax
from jax.experimental import pallas as pl
from jax.experimental.pallas import tpu as pltpu
from jax.experimental.pallas import tpu_sc as plsc

N = 10000
E = 160000
C = 256
CH = C // 2
K = 128
NCHUNK = 1344
NREAL = E // K
NEXTRA = NCHUNK - NREAL
EPAD = NCHUNK * K - E - N
NSUB = 16
NCORE = 2
CPT = NCHUNK // NSUB
CPW = NCHUNK // (NSUB * NCORE)
NACC = N + 512
NBUF = 2
ROWS_BIG, ROWS_SMALL = 632, 624

_mesh = lambda: plsc.VectorSubcoreMesh(core_axis_name="c", subcore_axis_name="s")


def _row_base(s):
    return ROWS_SMALL * s + 8 * jnp.minimum(s, 2)


@functools.partial(
    pl.kernel,
    out_type=jax.ShapeDtypeStruct((NCORE, 1, NACC), jnp.float32),
    mesh=_mesh(),
    scratch_types=[
        pltpu.VMEM((CPW, 1, K), jnp.int32),
        pltpu.VMEM((CPW, 1, K), jnp.int32),
        pltpu.VMEM((K,), jnp.float32),
        pltpu.VMEM((2000,), jnp.float32),
        pltpu.VMEM_SHARED((NACC,), jnp.float32),
        pltpu.SemaphoreType.DMA,
        pltpu.SemaphoreType.DMA,
    ],
)
def _deg_kernel(ed_hbm, ex_hbm, out_hbm, ed_slab, dst_slab, ones_v, zbuf, acc,
                semd, semz):
    c = lax.axis_index("c")
    s = lax.axis_index("s")
    w = s * NCORE + c

    base = w * CPW
    BW = NREAL // CPW
    RREM = NREAL - BW * CPW
    @pl.when(w < BW)
    def _():
        pltpu.sync_copy(ed_hbm.at[pl.ds(base, CPW)], ed_slab)
    @pl.when(w == BW)
    def _():
        pltpu.sync_copy(ed_hbm.at[pl.ds(BW * CPW, RREM)],
                        ed_slab.at[pl.ds(0, RREM)])
        pltpu.sync_copy(ex_hbm.at[pl.ds(0, CPW - RREM)],
                        ed_slab.at[pl.ds(RREM, CPW - RREM)])
    @pl.when(w > BW)
    def _():
        pltpu.sync_copy(ex_hbm.at[pl.ds(base - NREAL, CPW)], ed_slab)
    def unpack(i, _):
        sl = pl.ds((i % 8) * 16, 16)
        dst_slab[i // 8, 0, sl] = lax.shift_right_logical(ed_slab[i // 8, 0, sl], 14)
        return 0
    lax.fori_loop(0, CPW * (K // 16), unpack, 0)

    def fill_ones(i, _):
        ones_v[pl.ds(i * 16, 16)] = jnp.ones((16,), jnp.float32)
        return 0
    lax.fori_loop(0, K // 16, fill_ones, 0)
    def fill_z(i, _):
        zbuf[pl.ds(i * 16, 16)] = jnp.zeros((16,), jnp.float32)
        return 0
    lax.fori_loop(0, 125, fill_z, 0)
    @pl.when(s == 0)
    def _():
        def zfire(j, _):
            pltpu.async_copy(zbuf, acc.at[pl.ds(j * 2000, 2000)], semz)
            return 0
        lax.fori_loop(0, N // 2000, zfire, 0)
        def zdrain(j, _):
            pltpu.make_async_copy(zbuf, acc.at[pl.ds(0, 2000)], semz).wait()
            return 0
        lax.fori_loop(0, N // 2000, zdrain, 0)
    plsc.subcore_barrier()

    nch = CPW
    def fire(j, _):
        pltpu.async_copy(ones_v, acc.at[dst_slab.at[j, 0]], semd, add=True)
        return 0
    lax.fori_loop(0, nch, fire, 0)
    def drain(j, _):
        pltpu.make_async_copy(ones_v, acc.at[dst_slab.at[0, 0]], semd).wait()
        return 0
    lax.fori_loop(0, nch, drain, 0)
    plsc.subcore_barrier()

    @pl.when(s == 0)
    def _():
        pltpu.sync_copy(acc, out_hbm.at[c, 0])


@functools.partial(
    pl.kernel,
    out_type=jax.ShapeDtypeStruct((NCORE, N, CH), jnp.float32),
    mesh=_mesh(),
    scratch_types=[
        pltpu.VMEM((CPT, 1, K), jnp.int32),
        pltpu.VMEM((NBUF, 1, K), jnp.int32),
        pltpu.VMEM((NBUF, 1, K), jnp.int32),
        pltpu.VMEM((K, CH), jnp.float32),
        pltpu.VMEM((K, CH), jnp.float32),
        pltpu.VMEM((8, CH), jnp.float32),
        pltpu.VMEM_SHARED((NACC, CH), jnp.float32),
        pltpu.SemaphoreType.DMA,
        pltpu.SemaphoreType.DMA,
        pltpu.SemaphoreType.DMA,
        pltpu.SemaphoreType.DMA,
        pltpu.SemaphoreType.DMA,
    ],
)
def _agg_kernel(xs_hbm, ed_hbm, ex_hbm, out_hbm,
                ed_slab, src_v, dst_v, r0, r1, zbuf, acc,
                g0, g1, s0, s1, semz):
    rows = [r0, r1]
    semg = [g0, g1]
    sems = [s0, s1]
    c = lax.axis_index("c")
    s = lax.axis_index("s")

    base = s * CPT
    BT = NREAL // CPT
    TREM = NREAL - BT * CPT
    @pl.when(s < BT)
    def _():
        pltpu.sync_copy(ed_hbm.at[pl.ds(base, CPT)], ed_slab)
    @pl.when(s == BT)
    def _():
        pltpu.sync_copy(ed_hbm.at[pl.ds(BT * CPT, TREM)],
                        ed_slab.at[pl.ds(0, TREM)])
        pltpu.sync_copy(ex_hbm.at[pl.ds(0, CPT - TREM)],
                        ed_slab.at[pl.ds(TREM, CPT - TREM)])
    @pl.when(s > BT)
    def _():
        pltpu.sync_copy(ex_hbm.at[pl.ds(base - NREAL, CPT)], ed_slab)

    off = c * N
    def unpack(j, b):
        def go(i, _):
            sl = pl.ds(i * 16, 16)
            ed = ed_slab[j, 0, sl]
            src_v[b, 0, sl] = (ed & 0x3FFF) + off
            dst_v[b, 0, sl] = lax.shift_right_logical(ed, 14)
            return 0
        lax.fori_loop(0, K // 16, go, 0)

    def fill_z(i, _):
        zbuf[i // 8, pl.ds((i % 8) * 16, 16)] = jnp.zeros((16,), jnp.float32)
        return 0
    lax.fori_loop(0, 8 * (CH // 16), fill_z, 0)
    rbase = _row_base(s)
    nz = jnp.where(s < 2, ROWS_BIG // 8, ROWS_SMALL // 8)
    def zfire(j, _):
        pltpu.async_copy(zbuf, acc.at[pl.ds(rbase + j * 8, 8)], semz)
        return 0
    lax.fori_loop(0, nz, zfire, 0)
    def zdrain(j, _):
        pltpu.make_async_copy(zbuf, acc.at[pl.ds(rbase, 8)], semz).wait()
        return 0
    lax.fori_loop(0, nz, zdrain, 0)
    plsc.subcore_barrier()

    def gfire(j, b):
        pltpu.async_copy(xs_hbm.at[src_v.at[b, 0]], rows[b], semg[b])
    def gwait(b):
        pltpu.make_async_copy(xs_hbm.at[src_v.at[b, 0]], rows[b], semg[b]).wait()
    def sfire(b):
        pltpu.async_copy(rows[b], acc.at[dst_v.at[b, 0]], sems[b], add=True)
    def swait(b):
        pltpu.make_async_copy(rows[b], acc.at[dst_v.at[b, 0]], sems[b]).wait()

    unpack(0, 0)
    gfire(0, 0)
    unpack(1, 1)
    gfire(1, 1)
    gwait(0)
    sfire(0)
    def steady(g, _):
        for b in range(NBUF):
            j = g * NBUF + b
            swait(b)
            unpack(j, b)
            gfire(j, b)
            gwait(1 - b)
            sfire(1 - b)
        return 0
    lax.fori_loop(1, CPT // NBUF, steady, 0)
    gwait(1)
    sfire(1)
    swait(0)
    swait(1)
    plsc.subcore_barrier()

    rbig = pl.ds(rbase, ROWS_BIG)
    rsml = pl.ds(rbase, ROWS_SMALL)
    @pl.when(s < 2)
    def _():
        pltpu.sync_copy(acc.at[rbig], out_hbm.at[c, rbig])
    @pl.when(s >= 2)
    def _():
        pltpu.sync_copy(acc.at[rsml], out_hbm.at[c, rsml])


_TC_ROWS = 1000


def _trans_body(degp_ref, dis_ref):
    ident = jnp.eye(_TC_ROWS, dtype=jnp.float32)
    for chunk in range(N // _TC_ROWS):
        seg = (degp_ref[0, :, pl.ds(chunk * _TC_ROWS, _TC_ROWS)]
               + degp_ref[1, :, pl.ds(chunk * _TC_ROWS, _TC_ROWS)])
        col = lax.dot_general(ident, lax.rsqrt(seg),
                              (((1,), (1,)), ((), ())),
                              preferred_element_type=jnp.float32)
        dis_ref[pl.ds(chunk * _TC_ROWS, _TC_ROWS), :] = col


def _trans(degp):
    return pl.pallas_call(
        _trans_body,
        out_shape=jax.ShapeDtypeStruct((N, 1), jnp.float32),
    )(degp)


def _scale_body(dis_ref, x_ref, xs_ref):
    xs = x_ref[...] * dis_ref[...]
    xs_ref[0] = xs[:, :CH]
    xs_ref[1] = xs[:, CH:]


_SROWS = 2000


def _scale(dis, x):
    grid = N // _SROWS
    return pl.pallas_call(
        _scale_body,
        grid=(grid,),
        in_specs=[
            pl.BlockSpec((_SROWS, 1), lambda i: (i, 0)),
            pl.BlockSpec((_SROWS, C), lambda i: (i, 0)),
        ],
        out_specs=pl.BlockSpec((NCORE, _SROWS, CH), lambda i: (0, i, 0)),
        out_shape=jax.ShapeDtypeStruct((NCORE, N, CH), jnp.float32),
    )(dis, x)


def _final_body(dis_ref, zp_ref, wmu_ref, wls_ref, bmu_ref, bls_ref,
                mu_ref, ls_ref):
    dis = dis_ref[...]
    z = jnp.concatenate([dis * zp_ref[0], dis * zp_ref[1]],
                        axis=1).astype(jnp.bfloat16)
    mu_ref[...] = jnp.dot(z, wmu_ref[...].astype(jnp.bfloat16),
                          preferred_element_type=jnp.float32) + bmu_ref[...]
    ls_ref[...] = jnp.dot(z, wls_ref[...].astype(jnp.bfloat16),
                          preferred_element_type=jnp.float32) + bls_ref[...]


def _final(dis, zp, wmu, wls, bmu, bls):
    grid = N // _TC_ROWS
    return pl.pallas_call(
        _final_body,
        grid=(grid,),
        in_specs=[
            pl.BlockSpec((_TC_ROWS, 1), lambda i: (i, 0)),
            pl.BlockSpec((NCORE, _TC_ROWS, CH), lambda i: (0, i, 0)),
            pl.BlockSpec((C, C), lambda i: (0, 0)),
            pl.BlockSpec((C, C), lambda i: (0, 0)),
            pl.BlockSpec((1, C), lambda i: (0, 0)),
            pl.BlockSpec((1, C), lambda i: (0, 0)),
        ],
        out_specs=[
            pl.BlockSpec((_TC_ROWS, C), lambda i: (i, 0)),
            pl.BlockSpec((_TC_ROWS, C), lambda i: (i, 0)),
        ],
        out_shape=[
            jax.ShapeDtypeStruct((N, C), jnp.float32),
            jax.ShapeDtypeStruct((N, C), jnp.float32),
        ],
    )(dis, zp, wmu, wls, bmu, bls)


def kernel(x, edge_index, W_mu, b_mu, W_logstd, b_logstd):
    src = edge_index[0].astype(jnp.int32)
    dst = edge_index[1].astype(jnp.int32)
    ed = (src | (dst << 14)).reshape(NREAL, 1, K)
    iota = jnp.arange(N, dtype=jnp.int32)
    ipad = jnp.arange(EPAD, dtype=jnp.int32)
    ex = jnp.concatenate([
        iota | (iota << 14),
        (ipad & 0x1FFF) | ((N + (ipad & 511)) << 14),
    ]).reshape(NEXTRA, 1, K)

    degp = _deg_kernel(ed, ex)
    dis = _trans(degp)
    xs2 = _scale(dis, x)
    zp = _agg_kernel(xs2.reshape(2 * N, CH), ed, ex)
    mu, ls = _final(dis, zp, W_mu, W_logstd,
                    b_mu.reshape(1, C), b_logstd.reshape(1, C))
    return mu, ls


# speedup vs baseline: 1.7669x; 1.0111x over previous
import functools

import jax
import jax.numpy as jnp
from jax import lax
from jax.experimental import pallas as pl
from jax.experimental.pallas import tpu as pltpu
from jax.experimental.pallas import tpu_sc as plsc

N = 10000
E = 160000
C = 256
CH = C // 2
K = 128
NCHUNK = 1344
NREAL = E // K
NEXTRA = NCHUNK - NREAL
EPAD = NCHUNK * K - E - N
NSUB = 16
NCORE = 2
CPT = NCHUNK // NSUB
CPW = NCHUNK // (NSUB * NCORE)
NACC = N + 512
NBUF = 2
ROWS_BIG, ROWS_SMALL = 632, 624

_mesh = lambda: plsc.VectorSubcoreMesh(core_axis_name="c", subcore_axis_name="s")


def _row_base(s):
    return ROWS_SMALL * s + 8 * jnp.minimum(s, 2)


@functools.partial(
    pl.kernel,
    out_type=jax.ShapeDtypeStruct((NCORE, 1, NACC), jnp.float32),
    mesh=_mesh(),
    scratch_types=[
        pltpu.VMEM((CPW, 1, K), jnp.int32),
        pltpu.VMEM((CPW, 1, K), jnp.int32),
        pltpu.VMEM((K,), jnp.float32),
        pltpu.VMEM((2000,), jnp.float32),
        pltpu.VMEM_SHARED((NACC,), jnp.float32),
        pltpu.SemaphoreType.DMA,
        pltpu.SemaphoreType.DMA,
    ],
)
def _deg_kernel(ed_hbm, ex_hbm, out_hbm, ed_slab, dst_slab, ones_v, zbuf, acc,
                semd, semz):
    c = lax.axis_index("c")
    s = lax.axis_index("s")
    w = s * NCORE + c

    base = w * CPW
    BW = NREAL // CPW
    RREM = NREAL - BW * CPW
    @pl.when(w < BW)
    def _():
        pltpu.sync_copy(ed_hbm.at[pl.ds(base, CPW)], ed_slab)
    @pl.when(w == BW)
    def _():
        pltpu.sync_copy(ed_hbm.at[pl.ds(BW * CPW, RREM)],
                        ed_slab.at[pl.ds(0, RREM)])
        pltpu.sync_copy(ex_hbm.at[pl.ds(0, CPW - RREM)],
                        ed_slab.at[pl.ds(RREM, CPW - RREM)])
    @pl.when(w > BW)
    def _():
        pltpu.sync_copy(ex_hbm.at[pl.ds(base - NREAL, CPW)], ed_slab)
    def unpack(i, _):
        sl = pl.ds((i % 8) * 16, 16)
        dst_slab[i // 8, 0, sl] = lax.shift_right_logical(ed_slab[i // 8, 0, sl], 14)
        return 0
    lax.fori_loop(0, CPW * (K // 16), unpack, 0)

    def fill_ones(i, _):
        ones_v[pl.ds(i * 16, 16)] = jnp.ones((16,), jnp.float32)
        return 0
    lax.fori_loop(0, K // 16, fill_ones, 0)
    def fill_z(i, _):
        zbuf[pl.ds(i * 16, 16)] = jnp.zeros((16,), jnp.float32)
        return 0
    lax.fori_loop(0, 125, fill_z, 0)
    @pl.when(s == 0)
    def _():
        def zfire(j, _):
            pltpu.async_copy(zbuf, acc.at[pl.ds(j * 2000, 2000)], semz)
            return 0
        lax.fori_loop(0, N // 2000, zfire, 0)
        def zdrain(j, _):
            pltpu.make_async_copy(zbuf, acc.at[pl.ds(0, 2000)], semz).wait()
            return 0
        lax.fori_loop(0, N // 2000, zdrain, 0)
    plsc.subcore_barrier()

    nch = CPW
    def fire(j, _):
        pltpu.async_copy(ones_v, acc.at[dst_slab.at[j, 0]], semd, add=True)
        return 0
    lax.fori_loop(0, nch, fire, 0)
    def drain(j, _):
        pltpu.make_async_copy(ones_v, acc.at[dst_slab.at[0, 0]], semd).wait()
        return 0
    lax.fori_loop(0, nch, drain, 0)
    plsc.subcore_barrier()

    @pl.when(s == 0)
    def _():
        pltpu.sync_copy(acc, out_hbm.at[c, 0])


@functools.partial(
    pl.kernel,
    out_type=jax.ShapeDtypeStruct((NCORE, N, CH), jnp.float32),
    mesh=_mesh(),
    scratch_types=[
        pltpu.VMEM((CPT, 1, K), jnp.int32),
        pltpu.VMEM((NBUF, 1, K), jnp.int32),
        pltpu.VMEM((NBUF, 1, K), jnp.int32),
        pltpu.VMEM((K, CH), jnp.float32),
        pltpu.VMEM((K, CH), jnp.float32),
        pltpu.VMEM((8, CH), jnp.float32),
        pltpu.VMEM_SHARED((NACC, CH), jnp.float32),
        pltpu.SemaphoreType.DMA,
        pltpu.SemaphoreType.DMA,
        pltpu.SemaphoreType.DMA,
        pltpu.SemaphoreType.DMA,
        pltpu.SemaphoreType.DMA,
    ],
)
def _agg_kernel(xs_hbm, ed_hbm, ex_hbm, out_hbm,
                ed_slab, src_v, dst_v, r0, r1, zbuf, acc,
                g0, g1, s0, s1, semz):
    rows = [r0, r1]
    semg = [g0, g1]
    sems = [s0, s1]
    c = lax.axis_index("c")
    s = lax.axis_index("s")

    base = s * CPT
    BT = NREAL // CPT
    TREM = NREAL - BT * CPT
    @pl.when(s < BT)
    def _():
        pltpu.sync_copy(ed_hbm.at[pl.ds(base, CPT)], ed_slab)
    @pl.when(s == BT)
    def _():
        pltpu.sync_copy(ed_hbm.at[pl.ds(BT * CPT, TREM)],
                        ed_slab.at[pl.ds(0, TREM)])
        pltpu.sync_copy(ex_hbm.at[pl.ds(0, CPT - TREM)],
                        ed_slab.at[pl.ds(TREM, CPT - TREM)])
    @pl.when(s > BT)
    def _():
        pltpu.sync_copy(ex_hbm.at[pl.ds(base - NREAL, CPT)], ed_slab)

    off = c * N
    def unpack(j, b):
        def go(i, _):
            sl = pl.ds(i * 16, 16)
            ed = ed_slab[j, 0, sl]
            src_v[b, 0, sl] = (ed & 0x3FFF) + off
            dst_v[b, 0, sl] = lax.shift_right_logical(ed, 14)
            return 0
        lax.fori_loop(0, K // 16, go, 0)

    def fill_z(i, _):
        zbuf[i // 8, pl.ds((i % 8) * 16, 16)] = jnp.zeros((16,), jnp.float32)
        return 0
    lax.fori_loop(0, 8 * (CH // 16), fill_z, 0)
    rbase = _row_base(s)
    nz = jnp.where(s < 2, ROWS_BIG // 8, ROWS_SMALL // 8)
    def zfire(j, _):
        pltpu.async_copy(zbuf, acc.at[pl.ds(rbase + j * 8, 8)], semz)
        return 0
    lax.fori_loop(0, nz, zfire, 0)
    def zdrain(j, _):
        pltpu.make_async_copy(zbuf, acc.at[pl.ds(rbase, 8)], semz).wait()
        return 0
    lax.fori_loop(0, nz, zdrain, 0)
    plsc.subcore_barrier()

    def gfire(j, b):
        pltpu.async_copy(xs_hbm.at[src_v.at[b, 0]], rows[b], semg[b])
    def gwait(b):
        pltpu.make_async_copy(xs_hbm.at[src_v.at[b, 0]], rows[b], semg[b]).wait()
    def sfire(b):
        pltpu.async_copy(rows[b], acc.at[dst_v.at[b, 0]], sems[b], add=True)
    def swait(b):
        pltpu.make_async_copy(rows[b], acc.at[dst_v.at[b, 0]], sems[b]).wait()

    unpack(0, 0)
    gfire(0, 0)
    unpack(1, 1)
    gfire(1, 1)
    gwait(0)
    sfire(0)
    def steady(g, _):
        for b in range(NBUF):
            j = g * NBUF + b
            swait(b)
            unpack(j, b)
            gfire(j, b)
            gwait(1 - b)
            sfire(1 - b)
        return 0
    lax.fori_loop(1, CPT // NBUF, steady, 0)
    gwait(1)
    sfire(1)
    swait(0)
    swait(1)
    plsc.subcore_barrier()

    rbig = pl.ds(rbase, ROWS_BIG)
    rsml = pl.ds(rbase, ROWS_SMALL)
    @pl.when(s < 2)
    def _():
        pltpu.sync_copy(acc.at[rbig], out_hbm.at[c, rbig])
    @pl.when(s >= 2)
    def _():
        pltpu.sync_copy(acc.at[rsml], out_hbm.at[c, rsml])


_TC_ROWS = 1000


def _trans_body(degp_ref, dis_ref):
    ident = jnp.eye(_TC_ROWS, dtype=jnp.float32)
    for chunk in range(N // _TC_ROWS):
        seg = (degp_ref[0, :, pl.ds(chunk * _TC_ROWS, _TC_ROWS)]
               + degp_ref[1, :, pl.ds(chunk * _TC_ROWS, _TC_ROWS)])
        col = lax.dot_general(ident, lax.rsqrt(seg),
                              (((1,), (1,)), ((), ())),
                              preferred_element_type=jnp.float32)
        dis_ref[pl.ds(chunk * _TC_ROWS, _TC_ROWS), :] = col


def _trans(degp):
    return pl.pallas_call(
        _trans_body,
        out_shape=jax.ShapeDtypeStruct((N, 1), jnp.float32),
    )(degp)


def _scale_body(dis_ref, x_ref, xs_ref):
    xs = x_ref[...] * dis_ref[...]
    xs_ref[0] = xs[:, :CH]
    xs_ref[1] = xs[:, CH:]


_SROWS = 2000


def _scale(dis, x):
    grid = N // _SROWS
    return pl.pallas_call(
        _scale_body,
        grid=(grid,),
        in_specs=[
            pl.BlockSpec((_SROWS, 1), lambda i: (i, 0)),
            pl.BlockSpec((_SROWS, C), lambda i: (i, 0)),
        ],
        out_specs=pl.BlockSpec((NCORE, _SROWS, CH), lambda i: (0, i, 0)),
        out_shape=jax.ShapeDtypeStruct((NCORE, N, CH), jnp.float32),
    )(dis, x)


def _final_body(dis_ref, zp_ref, wmu_ref, wls_ref, bmu_ref, bls_ref,
                mu_ref, ls_ref):
    dis = dis_ref[...]
    z = jnp.concatenate([dis * zp_ref[0], dis * zp_ref[1]],
                        axis=1).astype(jnp.bfloat16)
    mu_ref[...] = jnp.dot(z, wmu_ref[...].astype(jnp.bfloat16),
                          preferred_element_type=jnp.float32) + bmu_ref[...]
    ls_ref[...] = jnp.dot(z, wls_ref[...].astype(jnp.bfloat16),
                          preferred_element_type=jnp.float32) + bls_ref[...]


def _final(dis, zp, wmu, wls, bmu, bls):
    grid = N // _SROWS
    return pl.pallas_call(
        _final_body,
        grid=(grid,),
        in_specs=[
            pl.BlockSpec((_SROWS, 1), lambda i: (i, 0)),
            pl.BlockSpec((NCORE, _SROWS, CH), lambda i: (0, i, 0)),
            pl.BlockSpec((C, C), lambda i: (0, 0)),
            pl.BlockSpec((C, C), lambda i: (0, 0)),
            pl.BlockSpec((1, C), lambda i: (0, 0)),
            pl.BlockSpec((1, C), lambda i: (0, 0)),
        ],
        out_specs=[
            pl.BlockSpec((_SROWS, C), lambda i: (i, 0)),
            pl.BlockSpec((_SROWS, C), lambda i: (i, 0)),
        ],
        out_shape=[
            jax.ShapeDtypeStruct((N, C), jnp.float32),
            jax.ShapeDtypeStruct((N, C), jnp.float32),
        ],
    )(dis, zp, wmu, wls, bmu, bls)


def kernel(x, edge_index, W_mu, b_mu, W_logstd, b_logstd):
    src = edge_index[0].astype(jnp.int32)
    dst = edge_index[1].astype(jnp.int32)
    ed = (src | (dst << 14)).reshape(NREAL, 1, K)
    iota = jnp.arange(N, dtype=jnp.int32)
    ipad = jnp.arange(EPAD, dtype=jnp.int32)
    ex = jnp.concatenate([
        iota | (iota << 14),
        (ipad & 0x1FFF) | ((N + (ipad & 511)) << 14),
    ]).reshape(NEXTRA, 1, K)

    degp = _deg_kernel(ed, ex)
    dis = _trans(degp)
    xs2 = _scale(dis, x)
    zp = _agg_kernel(xs2.reshape(2 * N, CH), ed, ex)
    mu, ls = _final(dis, zp, W_mu, W_logstd,
                    b_mu.reshape(1, C), b_logstd.reshape(1, C))
    return mu, ls
